# Initial kernel scaffold; baseline (speedup 1.0000x reference)
import jax
import jax.numpy as jnp
from jax.experimental import pallas as pl


def kernel(logits, target):
    raise NotImplementedError("write your pallas kernel here")


# speedup vs baseline: 29.9325x; 29.9325x over previous
import functools

import jax
import jax.numpy as jnp
from jax import lax
from jax.experimental import pallas as pl
from jax.experimental.pallas import tpu as pltpu
from jax.experimental.pallas import tpu_sc as plsc

N = 4 * 19 * 512 * 512
RIDX = min(int(0.001 * N), N - 1)
TGT_HI = N - RIDX

RA, CA = 19456, 1024
BA = 1024

NC, NS = 2, 16
NW = NC * NS
PER_TILE = N // NW
CHUNK = 8192
NCH = PER_TILE // CHUNK
HR, HC = 512, 128
HBINS = HR * HC

FR, FB, FC = 38912, 1024, 128


def _take16(v, idx):
    return lax.gather(
        v,
        idx[:, None],
        lax.GatherDimensionNumbers(
            offset_dims=(), collapsed_slice_dims=(0,), start_index_map=(0,)
        ),
        slice_sizes=(1,),
        mode=lax.GatherScatterMode.PROMISE_IN_BOUNDS,
    )


_SC_PARAMS = pltpu.CompilerParams(
    use_tc_tiling_on_sc=False, needs_layout_passes=False
)


def _mesh():
    return plsc.VectorSubcoreMesh(
        core_axis_name="c", subcore_axis_name="s", num_cores=NC, num_subcores=NS
    )


def _loss_body(x_ref, y_ref, o_ref):
    x = x_ref[...]
    y = y_ref[...]
    o_ref[...] = jnp.maximum(x, 0.0) - x * y + jnp.log1p(jnp.exp(-jnp.abs(x)))


_loss_call = pl.pallas_call(
    _loss_body,
    grid=(RA // BA,),
    in_specs=[
        pl.BlockSpec((BA, CA), lambda i: (i, 0)),
        pl.BlockSpec((BA, CA), lambda i: (i, 0)),
    ],
    out_specs=pl.BlockSpec((BA, CA), lambda i: (i, 0)),
    out_shape=jax.ShapeDtypeStruct((RA, CA), jnp.float32),
)


STR = HBINS // NS


def _zero_hist(hist_v):
    zero16 = jnp.zeros((16,), jnp.int32)

    def _z(i, _):
        hist_v[pl.ds(i * 16, 16)] = zero16
        return 0

    lax.fori_loop(0, HBINS // 16, _z, 0)


def _hist_hi_body(loss_ref, out_ref, data_v, hist_v):
    c = lax.axis_index("c")
    s = lax.axis_index("s")
    wid = s * NC + c
    _zero_hist(hist_v)

    ones = jnp.ones((16,), jnp.int32)
    base = wid * PER_TILE

    def _chunk(ci, _):
        pltpu.sync_copy(loss_ref.at[pl.ds(base + ci * CHUNK, CHUNK)], data_v)

        def _vec(j, _):
            v = data_v[pl.ds(j * 16, 16)]
            h = lax.shift_right_logical(v, 16)
            plsc.addupdate_scatter(hist_v, [h], ones)
            return 0

        return lax.fori_loop(0, CHUNK // 16, _vec, 0)

    lax.fori_loop(0, NCH, _chunk, 0)
    pltpu.sync_copy(hist_v, out_ref.at[wid])


_hist_hi = pl.kernel(
    _hist_hi_body,
    out_type=jax.ShapeDtypeStruct((NW, HBINS), jnp.int32),
    mesh=_mesh(),
    compiler_params=_SC_PARAMS,
    scratch_types=[
        pltpu.VMEM((CHUNK,), jnp.int32),
        pltpu.VMEM((HBINS,), jnp.int32),
    ],
)


def _hist_lo_body(loss_ref, outc_ref, out_ref, data_v, hist_v, meta_v):
    c = lax.axis_index("c")
    s = lax.axis_index("s")
    wid = s * NC + c
    _zero_hist(hist_v)
    pltpu.sync_copy(outc_ref.at[pl.ds(0, 16)], meta_v)
    b_vec = meta_v[pl.ds(0, 16)]

    ones = jnp.ones((16,), jnp.int32)
    base = wid * PER_TILE

    def _chunk(ci, _):
        pltpu.sync_copy(loss_ref.at[pl.ds(base + ci * CHUNK, CHUNK)], data_v)

        def _vec(j, _):
            v = data_v[pl.ds(j * 16, 16)]
            hi = lax.shift_right_logical(v, 16)
            lo = lax.bitwise_and(v, 0xFFFF)
            m = hi == b_vec
            plsc.addupdate_scatter(hist_v, [lo], ones, mask=m)
            return 0

        return lax.fori_loop(0, CHUNK // 16, _vec, 0)

    lax.fori_loop(0, NCH, _chunk, 0)
    pltpu.sync_copy(hist_v, out_ref.at[wid])


_hist_lo = pl.kernel(
    _hist_lo_body,
    out_type=jax.ShapeDtypeStruct((NW, HBINS), jnp.int32),
    mesh=_mesh(),
    compiler_params=_SC_PARAMS,
    scratch_types=[
        pltpu.VMEM((CHUNK,), jnp.int32),
        pltpu.VMEM((HBINS,), jnp.int32),
        pltpu.VMEM((16,), jnp.int32),
    ],
)


def _reduce_stripes(hists_ref, s, acc_v, tmp_v, shared):
    sb = s * STR
    pltpu.sync_copy(hists_ref.at[0, pl.ds(sb, STR)], acc_v)

    def _slot(k, _):
        pltpu.sync_copy(hists_ref.at[k, pl.ds(sb, STR)], tmp_v)

        def _add(j, _):
            acc_v[pl.ds(j * 16, 16)] += tmp_v[pl.ds(j * 16, 16)]
            return 0

        return lax.fori_loop(0, STR // 16, _add, 0)

    lax.fori_loop(1, NW, _slot, 0)
    pltpu.sync_copy(acc_v, shared.at[pl.ds(sb, STR)])


def _scan_combined(t_vec, tmp_v, shared):
    fifteen = jnp.full((16,), 15, jnp.int32)

    def _chunk(k, carry):
        pltpu.sync_copy(shared.at[pl.ds(k * STR, STR)], tmp_v)

        def _vec(j, carry2):
            total, found, b, pb, hb = carry2
            v = tmp_v[pl.ds(j * 16, 16)]
            sc = plsc.cumsum(v)
            cc = sc + total
            ge = cc >= t_vec
            npc = plsc.all_reduce_population_count(ge)
            anyv = npc > 0
            ffs = plsc.all_reduce_ffs(ge)
            idx_safe = jnp.minimum(ffs, fifteen)
            event = jnp.logical_and(anyv, jnp.logical_not(found))
            binbase = (k * (STR // 16) + j) * 16
            b = jnp.where(event, ffs + binbase, b)
            pb = jnp.where(event, _take16(cc, idx_safe), pb)
            hb = jnp.where(event, _take16(v, idx_safe), hb)
            found = jnp.logical_or(found, anyv)
            total = _take16(cc, fifteen)
            return total, found, b, pb, hb

        return lax.fori_loop(0, STR // 16, _vec, carry)

    zi = jnp.zeros((16,), jnp.int32)
    init = (zi, jnp.zeros((16,), jnp.bool_), zi, zi, zi)
    total, found, b, pb, hb = lax.fori_loop(0, NS, _chunk, init)
    return b, pb, hb


def _scan_hi_body(hists_ref, out_ref, acc_v, tmp_v, stage_v, shared):
    c = lax.axis_index("c")
    s = lax.axis_index("s")

    @pl.when(c == 0)
    def _():
        _reduce_stripes(hists_ref, s, acc_v, tmp_v, shared)

    plsc.subcore_barrier()

    @pl.when(jnp.logical_and(c == 0, s == 0))
    def _():
        t_vec = jnp.full((16,), TGT_HI, jnp.int32)
        b, pb, hb = _scan_combined(t_vec, tmp_v, shared)
        tp = hb - pb + t_vec
        stage_v[pl.ds(0, 16)] = b
        stage_v[pl.ds(16, 16)] = tp
        pltpu.sync_copy(stage_v, out_ref)


_scan_hi = pl.kernel(
    _scan_hi_body,
    out_type=jax.ShapeDtypeStruct((2 * 16,), jnp.int32),
    mesh=_mesh(),
    compiler_params=_SC_PARAMS,
    scratch_types=[
        pltpu.VMEM((STR,), jnp.int32),
        pltpu.VMEM((STR,), jnp.int32),
        pltpu.VMEM((2 * 16,), jnp.int32),
        pltpu.VMEM_SHARED((HBINS,), jnp.int32),
    ],
)


def _scan_lo_body(hists_ref, outc_ref, out_ref, acc_v, tmp_v, meta_v, stage_v, shared):
    c = lax.axis_index("c")
    s = lax.axis_index("s")

    @pl.when(c == 0)
    def _():
        _reduce_stripes(hists_ref, s, acc_v, tmp_v, shared)

    plsc.subcore_barrier()

    @pl.when(jnp.logical_and(c == 0, s == 0))
    def _():
        pltpu.sync_copy(outc_ref, meta_v)
        b_vec = meta_v[pl.ds(0, 16)]
        t_vec = meta_v[pl.ds(16, 16)]
        l, _pb, _hb = _scan_combined(t_vec, tmp_v, shared)
        bits = jnp.bitwise_or(jnp.left_shift(b_vec, 16), l)
        stage_v[pl.ds(0, 16)] = plsc.bitcast(bits, jnp.float32)
        pltpu.sync_copy(stage_v, out_ref)


_scan_lo = pl.kernel(
    _scan_lo_body,
    out_type=jax.ShapeDtypeStruct((16,), jnp.float32),
    mesh=_mesh(),
    compiler_params=_SC_PARAMS,
    scratch_types=[
        pltpu.VMEM((STR,), jnp.int32),
        pltpu.VMEM((STR,), jnp.int32),
        pltpu.VMEM((2 * 16,), jnp.int32),
        pltpu.VMEM((16,), jnp.float32),
        pltpu.VMEM_SHARED((HBINS,), jnp.int32),
    ],
)


def _reduce_body(t_ref, loss_ref, o_ref, acc_s, acc_c):
    i = pl.program_id(0)
    t = t_ref[0, 0]
    blk = loss_ref[...]
    m = blk > t
    ps = jnp.sum(jnp.where(m, blk, 0.0), axis=1)
    pc = jnp.sum(m.astype(jnp.float32), axis=1)

    @pl.when(i == 0)
    def _():
        acc_s[...] = ps
        acc_c[...] = pc

    @pl.when(i > 0)
    def _():
        acc_s[...] += ps
        acc_c[...] += pc

    @pl.when(i == pl.num_programs(0) - 1)
    def _():
        ts = jnp.sum(acc_s[...], axis=1, keepdims=True)
        tc = jnp.sum(acc_c[...], axis=1, keepdims=True)
        o_ref[...] = jnp.broadcast_to(ts / tc, (4, HC))


_reduce_call = pl.pallas_call(
    _reduce_body,
    grid=(FR // FB,),
    in_specs=[
        pl.BlockSpec(memory_space=pltpu.SMEM),
        pl.BlockSpec((4, FB, FC), lambda i: (0, i, 0)),
    ],
    out_specs=pl.BlockSpec((4, HC), lambda i: (0, 0)),
    out_shape=jax.ShapeDtypeStruct((4, HC), jnp.float32),
    scratch_shapes=[
        pltpu.VMEM((4, HC), jnp.float32),
        pltpu.VMEM((4, HC), jnp.float32),
    ],
)


def kernel(logits, target):
    x = logits.reshape(RA, CA)
    y = target.reshape(RA, CA)
    loss = _loss_call(x, y)
    li = lax.bitcast_convert_type(loss, jnp.int32).reshape(N)
    hh = _hist_hi(li)
    outc = _scan_hi(hh)
    hl = _hist_lo(li, outc)
    oute = _scan_lo(hl, outc)
    t_s = oute[0].reshape(1, 1)
    om = _reduce_call(t_s, loss.reshape(4, FR, FC))
    return om[:, 0]


# speedup vs baseline: 34.9087x; 1.1662x over previous
import functools

import jax
import jax.numpy as jnp
from jax import lax
from jax.experimental import pallas as pl
from jax.experimental.pallas import tpu as pltpu
from jax.experimental.pallas import tpu_sc as plsc

N = 4 * 19 * 512 * 512
RIDX = min(int(0.001 * N), N - 1)
TGT_HI = N - RIDX

RA, CA = 19456, 1024
BA = 1024

NC, NS = 2, 16
NW = NC * NS
PER_TILE = N // NW
CHUNK = 8192
NCH = PER_TILE // CHUNK
HR, HC = 512, 128
HBINS = HR * HC

FR, FB, FC = 38912, 1024, 128


def _take16(v, idx):
    return lax.gather(
        v,
        idx[:, None],
        lax.GatherDimensionNumbers(
            offset_dims=(), collapsed_slice_dims=(0,), start_index_map=(0,)
        ),
        slice_sizes=(1,),
        mode=lax.GatherScatterMode.PROMISE_IN_BOUNDS,
    )


_SC_PARAMS = pltpu.CompilerParams(
    use_tc_tiling_on_sc=False, needs_layout_passes=False
)


def _mesh():
    return plsc.VectorSubcoreMesh(
        core_axis_name="c", subcore_axis_name="s", num_cores=NC, num_subcores=NS
    )


def _loss_body(x_ref, y_ref, o_ref):
    x = x_ref[...]
    y = y_ref[...]
    o_ref[...] = jnp.maximum(x, 0.0) - x * y + jnp.log1p(jnp.exp(-jnp.abs(x)))


_loss_call = pl.pallas_call(
    _loss_body,
    grid=(RA // BA,),
    in_specs=[
        pl.BlockSpec((BA, CA), lambda i: (i, 0)),
        pl.BlockSpec((BA, CA), lambda i: (i, 0)),
    ],
    out_specs=pl.BlockSpec((BA, CA), lambda i: (i, 0)),
    out_shape=jax.ShapeDtypeStruct((RA, CA), jnp.float32),
)


STR = HBINS // NS


def _zero_hist(hist_v):
    zero16 = jnp.zeros((16,), jnp.int32)

    def _z(i, _):
        hist_v[pl.ds(i * 16, 16)] = zero16
        return 0

    lax.fori_loop(0, HBINS // 16, _z, 0)


def _hist_stream(loss_ref, base, d0, d1, sem0, sem1, process):
    pltpu.async_copy(loss_ref.at[pl.ds(base, CHUNK)], d0, sem0)
    pltpu.async_copy(loss_ref.at[pl.ds(base + CHUNK, CHUNK)], d1, sem1)

    def _pair(g, _):
        off = base + 2 * g * CHUNK
        pltpu.make_async_copy(loss_ref.at[pl.ds(off, CHUNK)], d0, sem0).wait()
        process(d0)
        pltpu.async_copy(loss_ref.at[pl.ds(off + 2 * CHUNK, CHUNK)], d0, sem0)
        pltpu.make_async_copy(
            loss_ref.at[pl.ds(off + CHUNK, CHUNK)], d1, sem1
        ).wait()
        process(d1)
        pltpu.async_copy(loss_ref.at[pl.ds(off + 3 * CHUNK, CHUNK)], d1, sem1)
        return 0

    lax.fori_loop(0, NCH // 2 - 1, _pair, 0)
    tail = base + (NCH - 2) * CHUNK
    pltpu.make_async_copy(loss_ref.at[pl.ds(tail, CHUNK)], d0, sem0).wait()
    process(d0)
    pltpu.make_async_copy(loss_ref.at[pl.ds(tail + CHUNK, CHUNK)], d1, sem1).wait()
    process(d1)


def _hist_hi_body(loss_ref, out_ref, d0, d1, hist_v, sem0, sem1):
    c = lax.axis_index("c")
    s = lax.axis_index("s")
    wid = s * NC + c
    _zero_hist(hist_v)

    ones = jnp.ones((16,), jnp.int32)

    def _process(buf):
        def _vec(j, _):
            v0 = plsc.bitcast(buf[pl.ds(j * 32, 16)], jnp.int32)
            plsc.addupdate_scatter(
                hist_v, [lax.shift_right_logical(v0, 16)], ones
            )
            v1 = plsc.bitcast(buf[pl.ds(j * 32 + 16, 16)], jnp.int32)
            plsc.addupdate_scatter(
                hist_v, [lax.shift_right_logical(v1, 16)], ones
            )
            return 0

        lax.fori_loop(0, CHUNK // 32, _vec, 0)

    _hist_stream(loss_ref, wid * PER_TILE, d0, d1, sem0, sem1, _process)
    pltpu.sync_copy(hist_v, out_ref.at[wid])


_hist_hi = pl.kernel(
    _hist_hi_body,
    out_type=jax.ShapeDtypeStruct((NW, HBINS), jnp.int32),
    mesh=_mesh(),
    compiler_params=_SC_PARAMS,
    scratch_types=[
        pltpu.VMEM((CHUNK,), jnp.float32),
        pltpu.VMEM((CHUNK,), jnp.float32),
        pltpu.VMEM((HBINS,), jnp.int32),
        pltpu.SemaphoreType.DMA,
        pltpu.SemaphoreType.DMA,
    ],
)


def _hist_lo_body(loss_ref, outc_ref, out_ref, d0, d1, hist_v, meta_v, sem0, sem1):
    c = lax.axis_index("c")
    s = lax.axis_index("s")
    wid = s * NC + c
    _zero_hist(hist_v)
    pltpu.sync_copy(outc_ref.at[pl.ds(0, 16)], meta_v)
    b_vec = meta_v[pl.ds(0, 16)]

    ones = jnp.ones((16,), jnp.int32)

    def _process(buf):
        def _vec(j, _):
            for u in range(2):
                v = plsc.bitcast(buf[pl.ds(j * 32 + u * 16, 16)], jnp.int32)
                hi = lax.shift_right_logical(v, 16)
                lo = lax.bitwise_and(v, 0xFFFF)
                plsc.addupdate_scatter(hist_v, [lo], ones, mask=hi == b_vec)
            return 0

        lax.fori_loop(0, CHUNK // 32, _vec, 0)

    _hist_stream(loss_ref, wid * PER_TILE, d0, d1, sem0, sem1, _process)
    pltpu.sync_copy(hist_v, out_ref.at[wid])


_hist_lo = pl.kernel(
    _hist_lo_body,
    out_type=jax.ShapeDtypeStruct((NW, HBINS), jnp.int32),
    mesh=_mesh(),
    compiler_params=_SC_PARAMS,
    scratch_types=[
        pltpu.VMEM((CHUNK,), jnp.float32),
        pltpu.VMEM((CHUNK,), jnp.float32),
        pltpu.VMEM((HBINS,), jnp.int32),
        pltpu.VMEM((16,), jnp.int32),
        pltpu.SemaphoreType.DMA,
        pltpu.SemaphoreType.DMA,
    ],
)


def _reduce_stripes(hists_ref, s, acc_v, tmp_v, shared):
    sb = s * STR
    pltpu.sync_copy(hists_ref.at[0, pl.ds(sb, STR)], acc_v)

    def _slot(k, _):
        pltpu.sync_copy(hists_ref.at[k, pl.ds(sb, STR)], tmp_v)

        def _add(j, _):
            acc_v[pl.ds(j * 16, 16)] += tmp_v[pl.ds(j * 16, 16)]
            return 0

        return lax.fori_loop(0, STR // 16, _add, 0)

    lax.fori_loop(1, NW, _slot, 0)
    pltpu.sync_copy(acc_v, shared.at[pl.ds(sb, STR)])


def _scan_combined(t_vec, tmp_v, shared):
    fifteen = jnp.full((16,), 15, jnp.int32)

    def _chunk(k, carry):
        pltpu.sync_copy(shared.at[pl.ds(k * STR, STR)], tmp_v)

        def _vec(j, carry2):
            total, found, b, pb, hb = carry2
            v = tmp_v[pl.ds(j * 16, 16)]
            sc = plsc.cumsum(v)
            cc = sc + total
            ge = cc >= t_vec
            npc = plsc.all_reduce_population_count(ge)
            anyv = npc > 0
            ffs = plsc.all_reduce_ffs(ge)
            idx_safe = jnp.minimum(ffs, fifteen)
            event = jnp.logical_and(anyv, jnp.logical_not(found))
            binbase = (k * (STR // 16) + j) * 16
            b = jnp.where(event, ffs + binbase, b)
            pb = jnp.where(event, _take16(cc, idx_safe), pb)
            hb = jnp.where(event, _take16(v, idx_safe), hb)
            found = jnp.logical_or(found, anyv)
            total = _take16(cc, fifteen)
            return total, found, b, pb, hb

        return lax.fori_loop(0, STR // 16, _vec, carry)

    zi = jnp.zeros((16,), jnp.int32)
    init = (zi, jnp.zeros((16,), jnp.bool_), zi, zi, zi)
    total, found, b, pb, hb = lax.fori_loop(0, NS, _chunk, init)
    return b, pb, hb


def _scan_hi_body(hists_ref, out_ref, acc_v, tmp_v, stage_v, shared):
    c = lax.axis_index("c")
    s = lax.axis_index("s")

    @pl.when(c == 0)
    def _():
        _reduce_stripes(hists_ref, s, acc_v, tmp_v, shared)

    plsc.subcore_barrier()

    @pl.when(jnp.logical_and(c == 0, s == 0))
    def _():
        t_vec = jnp.full((16,), TGT_HI, jnp.int32)
        b, pb, hb = _scan_combined(t_vec, tmp_v, shared)
        tp = hb - pb + t_vec
        stage_v[pl.ds(0, 16)] = b
        stage_v[pl.ds(16, 16)] = tp
        pltpu.sync_copy(stage_v, out_ref)


_scan_hi = pl.kernel(
    _scan_hi_body,
    out_type=jax.ShapeDtypeStruct((2 * 16,), jnp.int32),
    mesh=_mesh(),
    compiler_params=_SC_PARAMS,
    scratch_types=[
        pltpu.VMEM((STR,), jnp.int32),
        pltpu.VMEM((STR,), jnp.int32),
        pltpu.VMEM((2 * 16,), jnp.int32),
        pltpu.VMEM_SHARED((HBINS,), jnp.int32),
    ],
)


def _scan_lo_body(hists_ref, outc_ref, out_ref, acc_v, tmp_v, meta_v, stage_v, shared):
    c = lax.axis_index("c")
    s = lax.axis_index("s")

    @pl.when(c == 0)
    def _():
        _reduce_stripes(hists_ref, s, acc_v, tmp_v, shared)

    plsc.subcore_barrier()

    @pl.when(jnp.logical_and(c == 0, s == 0))
    def _():
        pltpu.sync_copy(outc_ref, meta_v)
        b_vec = meta_v[pl.ds(0, 16)]
        t_vec = meta_v[pl.ds(16, 16)]
        l, _pb, _hb = _scan_combined(t_vec, tmp_v, shared)
        bits = jnp.bitwise_or(jnp.left_shift(b_vec, 16), l)
        stage_v[pl.ds(0, 16)] = plsc.bitcast(bits, jnp.float32)
        pltpu.sync_copy(stage_v, out_ref)


_scan_lo = pl.kernel(
    _scan_lo_body,
    out_type=jax.ShapeDtypeStruct((16,), jnp.float32),
    mesh=_mesh(),
    compiler_params=_SC_PARAMS,
    scratch_types=[
        pltpu.VMEM((STR,), jnp.int32),
        pltpu.VMEM((STR,), jnp.int32),
        pltpu.VMEM((2 * 16,), jnp.int32),
        pltpu.VMEM((16,), jnp.float32),
        pltpu.VMEM_SHARED((HBINS,), jnp.int32),
    ],
)


def _reduce_body(t_ref, loss_ref, o_ref, acc_s, acc_c):
    i = pl.program_id(0)
    t = t_ref[0, 0]
    blk = loss_ref[...]
    m = blk > t
    ps = jnp.sum(jnp.where(m, blk, 0.0), axis=1)
    pc = jnp.sum(m.astype(jnp.float32), axis=1)

    @pl.when(i == 0)
    def _():
        acc_s[...] = ps
        acc_c[...] = pc

    @pl.when(i > 0)
    def _():
        acc_s[...] += ps
        acc_c[...] += pc

    @pl.when(i == pl.num_programs(0) - 1)
    def _():
        ts = jnp.sum(acc_s[...], axis=1, keepdims=True)
        tc = jnp.sum(acc_c[...], axis=1, keepdims=True)
        o_ref[...] = jnp.broadcast_to(ts / tc, (4, HC))


_reduce_call = pl.pallas_call(
    _reduce_body,
    grid=(FR // FB,),
    in_specs=[
        pl.BlockSpec(memory_space=pltpu.SMEM),
        pl.BlockSpec((4, FB, FC), lambda i: (0, i, 0)),
    ],
    out_specs=pl.BlockSpec((4, HC), lambda i: (0, 0)),
    out_shape=jax.ShapeDtypeStruct((4, HC), jnp.float32),
    scratch_shapes=[
        pltpu.VMEM((4, HC), jnp.float32),
        pltpu.VMEM((4, HC), jnp.float32),
    ],
)


def kernel(logits, target):
    x = logits.reshape(RA, CA)
    y = target.reshape(RA, CA)
    loss = _loss_call(x, y)
    lf = loss.reshape(N)
    hh = _hist_hi(lf)
    outc = _scan_hi(hh)
    hl = _hist_lo(lf, outc)
    oute = _scan_lo(hl, outc)
    t_s = oute[0].reshape(1, 1)
    om = _reduce_call(t_s, loss.reshape(4, FR, FC))
    return om[:, 0]


# speedup vs baseline: 58.4414x; 1.6741x over previous
import functools

import jax
import jax.numpy as jnp
from jax import lax
from jax.experimental import pallas as pl
from jax.experimental.pallas import tpu as pltpu
from jax.experimental.pallas import tpu_sc as plsc

N = 4 * 19 * 512 * 512
RIDX = min(int(0.001 * N), N - 1)
TGT_HI = N - RIDX

RA, CA = 19456, 1024
BA = 1024

NC, NS = 2, 16
NW = NC * NS
PER_TILE = N // NW
CHUNK = 8192
NCH = PER_TILE // CHUNK
HR, HC = 512, 128
HBINS = HR * HC

FR, FB, FC = 38912, 1024, 128


def _take16(v, idx):
    return lax.gather(
        v,
        idx[:, None],
        lax.GatherDimensionNumbers(
            offset_dims=(), collapsed_slice_dims=(0,), start_index_map=(0,)
        ),
        slice_sizes=(1,),
        mode=lax.GatherScatterMode.PROMISE_IN_BOUNDS,
    )


_SC_PARAMS = pltpu.CompilerParams(
    use_tc_tiling_on_sc=False, needs_layout_passes=False
)


def _mesh():
    return plsc.VectorSubcoreMesh(
        core_axis_name="c", subcore_axis_name="s", num_cores=NC, num_subcores=NS
    )


def _loss_body(x_ref, y_ref, o_ref):
    x = x_ref[...]
    y = y_ref[...]
    o_ref[...] = jnp.maximum(x, 0.0) - x * y + jnp.log1p(jnp.exp(-jnp.abs(x)))


_loss_call = pl.pallas_call(
    _loss_body,
    grid=(RA // BA,),
    in_specs=[
        pl.BlockSpec((BA, CA), lambda i: (i, 0)),
        pl.BlockSpec((BA, CA), lambda i: (i, 0)),
    ],
    out_specs=pl.BlockSpec((BA, CA), lambda i: (i, 0)),
    out_shape=jax.ShapeDtypeStruct((RA, CA), jnp.float32),
)


STR = HBINS // NS


def _zero_hist(hist_v):
    zero16 = jnp.zeros((16,), jnp.int32)

    def _z(i, _):
        hist_v[pl.ds(i * 16, 16)] = zero16
        return 0

    lax.fori_loop(0, HBINS // 16, _z, 0)


def _hist_stream(loss_ref, base, d0, d1, sem0, sem1, process):
    pltpu.async_copy(loss_ref.at[pl.ds(base, CHUNK)], d0, sem0)
    pltpu.async_copy(loss_ref.at[pl.ds(base + CHUNK, CHUNK)], d1, sem1)

    def _pair(g, _):
        off = base + 2 * g * CHUNK
        pltpu.make_async_copy(loss_ref.at[pl.ds(off, CHUNK)], d0, sem0).wait()
        process(d0)
        pltpu.async_copy(loss_ref.at[pl.ds(off + 2 * CHUNK, CHUNK)], d0, sem0)
        pltpu.make_async_copy(
            loss_ref.at[pl.ds(off + CHUNK, CHUNK)], d1, sem1
        ).wait()
        process(d1)
        pltpu.async_copy(loss_ref.at[pl.ds(off + 3 * CHUNK, CHUNK)], d1, sem1)
        return 0

    lax.fori_loop(0, NCH // 2 - 1, _pair, 0)
    tail = base + (NCH - 2) * CHUNK
    pltpu.make_async_copy(loss_ref.at[pl.ds(tail, CHUNK)], d0, sem0).wait()
    process(d0)
    pltpu.make_async_copy(loss_ref.at[pl.ds(tail + CHUNK, CHUNK)], d1, sem1).wait()
    process(d1)


def _hist_hi_body(loss_ref, out_ref, d0, d1, hist_v, sem0, sem1):
    c = lax.axis_index("c")
    s = lax.axis_index("s")
    wid = s * NC + c
    _zero_hist(hist_v)

    ones = jnp.ones((16,), jnp.int32)

    def _process(buf):
        @plsc.parallel_loop(0, CHUNK // 16, unroll=8)
        def _vec(j):
            v = plsc.bitcast(buf[pl.ds(j * 16, 16)], jnp.int32)
            plsc.addupdate_scatter(
                hist_v, [lax.shift_right_logical(v, 16)], ones
            )

    _hist_stream(loss_ref, wid * PER_TILE, d0, d1, sem0, sem1, _process)
    pltpu.sync_copy(hist_v, out_ref.at[wid])


_hist_hi = pl.kernel(
    _hist_hi_body,
    out_type=jax.ShapeDtypeStruct((NW, HBINS), jnp.int32),
    mesh=_mesh(),
    compiler_params=_SC_PARAMS,
    scratch_types=[
        pltpu.VMEM((CHUNK,), jnp.float32),
        pltpu.VMEM((CHUNK,), jnp.float32),
        pltpu.VMEM((HBINS,), jnp.int32),
        pltpu.SemaphoreType.DMA,
        pltpu.SemaphoreType.DMA,
    ],
)


def _hist_lo_body(loss_ref, outc_ref, out_ref, d0, d1, hist_v, meta_v, sem0, sem1):
    c = lax.axis_index("c")
    s = lax.axis_index("s")
    wid = s * NC + c
    _zero_hist(hist_v)
    pltpu.sync_copy(outc_ref.at[pl.ds(0, 16)], meta_v)
    b_vec = meta_v[pl.ds(0, 16)]

    ones = jnp.ones((16,), jnp.int32)

    def _process(buf):
        @plsc.parallel_loop(0, CHUNK // 16, unroll=8)
        def _vec(j):
            v = plsc.bitcast(buf[pl.ds(j * 16, 16)], jnp.int32)
            hi = lax.shift_right_logical(v, 16)
            lo = lax.bitwise_and(v, 0xFFFF)
            plsc.addupdate_scatter(hist_v, [lo], ones, mask=hi == b_vec)

    _hist_stream(loss_ref, wid * PER_TILE, d0, d1, sem0, sem1, _process)
    pltpu.sync_copy(hist_v, out_ref.at[wid])


_hist_lo = pl.kernel(
    _hist_lo_body,
    out_type=jax.ShapeDtypeStruct((NW, HBINS), jnp.int32),
    mesh=_mesh(),
    compiler_params=_SC_PARAMS,
    scratch_types=[
        pltpu.VMEM((CHUNK,), jnp.float32),
        pltpu.VMEM((CHUNK,), jnp.float32),
        pltpu.VMEM((HBINS,), jnp.int32),
        pltpu.VMEM((16,), jnp.int32),
        pltpu.SemaphoreType.DMA,
        pltpu.SemaphoreType.DMA,
    ],
)


def _reduce_stripes(hists_ref, s, acc_v, tmp_v, shared):
    sb = s * STR
    pltpu.sync_copy(hists_ref.at[0, pl.ds(sb, STR)], acc_v)

    def _slot(k, _):
        pltpu.sync_copy(hists_ref.at[k, pl.ds(sb, STR)], tmp_v)

        def _add(j, _):
            acc_v[pl.ds(j * 16, 16)] += tmp_v[pl.ds(j * 16, 16)]
            return 0

        return lax.fori_loop(0, STR // 16, _add, 0)

    lax.fori_loop(1, NW, _slot, 0)
    pltpu.sync_copy(acc_v, shared.at[pl.ds(sb, STR)])


def _scan_combined(t_vec, tmp_v, shared):
    fifteen = jnp.full((16,), 15, jnp.int32)

    def _chunk(k, carry):
        pltpu.sync_copy(shared.at[pl.ds(k * STR, STR)], tmp_v)

        def _vec(j, carry2):
            total, found, b, pb, hb = carry2
            v = tmp_v[pl.ds(j * 16, 16)]
            sc = plsc.cumsum(v)
            cc = sc + total
            ge = cc >= t_vec
            npc = plsc.all_reduce_population_count(ge)
            anyv = npc > 0
            ffs = plsc.all_reduce_ffs(ge)
            idx_safe = jnp.minimum(ffs, fifteen)
            event = jnp.logical_and(anyv, jnp.logical_not(found))
            binbase = (k * (STR // 16) + j) * 16
            b = jnp.where(event, ffs + binbase, b)
            pb = jnp.where(event, _take16(cc, idx_safe), pb)
            hb = jnp.where(event, _take16(v, idx_safe), hb)
            found = jnp.logical_or(found, anyv)
            total = _take16(cc, fifteen)
            return total, found, b, pb, hb

        return lax.fori_loop(0, STR // 16, _vec, carry)

    zi = jnp.zeros((16,), jnp.int32)
    init = (zi, jnp.zeros((16,), jnp.bool_), zi, zi, zi)
    total, found, b, pb, hb = lax.fori_loop(0, NS, _chunk, init)
    return b, pb, hb


def _scan_hi_body(hists_ref, out_ref, acc_v, tmp_v, stage_v, shared):
    c = lax.axis_index("c")
    s = lax.axis_index("s")

    @pl.when(c == 0)
    def _():
        _reduce_stripes(hists_ref, s, acc_v, tmp_v, shared)

    plsc.subcore_barrier()

    @pl.when(jnp.logical_and(c == 0, s == 0))
    def _():
        t_vec = jnp.full((16,), TGT_HI, jnp.int32)
        b, pb, hb = _scan_combined(t_vec, tmp_v, shared)
        tp = hb - pb + t_vec
        stage_v[pl.ds(0, 16)] = b
        stage_v[pl.ds(16, 16)] = tp
        pltpu.sync_copy(stage_v, out_ref)


_scan_hi = pl.kernel(
    _scan_hi_body,
    out_type=jax.ShapeDtypeStruct((2 * 16,), jnp.int32),
    mesh=_mesh(),
    compiler_params=_SC_PARAMS,
    scratch_types=[
        pltpu.VMEM((STR,), jnp.int32),
        pltpu.VMEM((STR,), jnp.int32),
        pltpu.VMEM((2 * 16,), jnp.int32),
        pltpu.VMEM_SHARED((HBINS,), jnp.int32),
    ],
)


def _scan_lo_body(hists_ref, outc_ref, out_ref, acc_v, tmp_v, meta_v, stage_v, shared):
    c = lax.axis_index("c")
    s = lax.axis_index("s")

    @pl.when(c == 0)
    def _():
        _reduce_stripes(hists_ref, s, acc_v, tmp_v, shared)

    plsc.subcore_barrier()

    @pl.when(jnp.logical_and(c == 0, s == 0))
    def _():
        pltpu.sync_copy(outc_ref, meta_v)
        b_vec = meta_v[pl.ds(0, 16)]
        t_vec = meta_v[pl.ds(16, 16)]
        l, _pb, _hb = _scan_combined(t_vec, tmp_v, shared)
        bits = jnp.bitwise_or(jnp.left_shift(b_vec, 16), l)
        stage_v[pl.ds(0, 16)] = plsc.bitcast(bits, jnp.float32)
        pltpu.sync_copy(stage_v, out_ref)


_scan_lo = pl.kernel(
    _scan_lo_body,
    out_type=jax.ShapeDtypeStruct((16,), jnp.float32),
    mesh=_mesh(),
    compiler_params=_SC_PARAMS,
    scratch_types=[
        pltpu.VMEM((STR,), jnp.int32),
        pltpu.VMEM((STR,), jnp.int32),
        pltpu.VMEM((2 * 16,), jnp.int32),
        pltpu.VMEM((16,), jnp.float32),
        pltpu.VMEM_SHARED((HBINS,), jnp.int32),
    ],
)


def _reduce_body(t_ref, loss_ref, o_ref, acc_s, acc_c):
    i = pl.program_id(0)
    t = t_ref[0, 0]
    blk = loss_ref[...]
    m = blk > t
    ps = jnp.sum(jnp.where(m, blk, 0.0), axis=1)
    pc = jnp.sum(m.astype(jnp.float32), axis=1)

    @pl.when(i == 0)
    def _():
        acc_s[...] = ps
        acc_c[...] = pc

    @pl.when(i > 0)
    def _():
        acc_s[...] += ps
        acc_c[...] += pc

    @pl.when(i == pl.num_programs(0) - 1)
    def _():
        ts = jnp.sum(acc_s[...], axis=1, keepdims=True)
        tc = jnp.sum(acc_c[...], axis=1, keepdims=True)
        o_ref[...] = jnp.broadcast_to(ts / tc, (4, HC))


_reduce_call = pl.pallas_call(
    _reduce_body,
    grid=(FR // FB,),
    in_specs=[
        pl.BlockSpec(memory_space=pltpu.SMEM),
        pl.BlockSpec((4, FB, FC), lambda i: (0, i, 0)),
    ],
    out_specs=pl.BlockSpec((4, HC), lambda i: (0, 0)),
    out_shape=jax.ShapeDtypeStruct((4, HC), jnp.float32),
    scratch_shapes=[
        pltpu.VMEM((4, HC), jnp.float32),
        pltpu.VMEM((4, HC), jnp.float32),
    ],
)


def kernel(logits, target):
    x = logits.reshape(RA, CA)
    y = target.reshape(RA, CA)
    loss = _loss_call(x, y)
    lf = loss.reshape(N)
    hh = _hist_hi(lf)
    outc = _scan_hi(hh)
    hl = _hist_lo(lf, outc)
    oute = _scan_lo(hl, outc)
    t_s = oute[0].reshape(1, 1)
    om = _reduce_call(t_s, loss.reshape(4, FR, FC))
    return om[:, 0]


# speedup vs baseline: 58.4822x; 1.0007x over previous
import functools

import jax
import jax.numpy as jnp
from jax import lax
from jax.experimental import pallas as pl
from jax.experimental.pallas import tpu as pltpu
from jax.experimental.pallas import tpu_sc as plsc

N = 4 * 19 * 512 * 512
RIDX = min(int(0.001 * N), N - 1)
TGT_HI = N - RIDX

RA, CA = 19456, 1024
BA = 1024

NC, NS = 2, 16
NW = NC * NS
PER_TILE = N // NW
TROWS = RA // NW
CROWS = 8
CHUNK = CROWS * CA
NCH = TROWS // CROWS
HR, HC = 512, 128
HBINS = HR * HC

FR, FB, FC = 38912, 1024, 128


def _take16(v, idx):
    return lax.gather(
        v,
        idx[:, None],
        lax.GatherDimensionNumbers(
            offset_dims=(), collapsed_slice_dims=(0,), start_index_map=(0,)
        ),
        slice_sizes=(1,),
        mode=lax.GatherScatterMode.PROMISE_IN_BOUNDS,
    )


_SC_PARAMS = pltpu.CompilerParams(
    use_tc_tiling_on_sc=False, needs_layout_passes=False
)


def _mesh():
    return plsc.VectorSubcoreMesh(
        core_axis_name="c", subcore_axis_name="s", num_cores=NC, num_subcores=NS
    )


def _loss_body(x_ref, y_ref, o_ref):
    x = x_ref[...]
    y = y_ref[...]
    o_ref[...] = jnp.maximum(x, 0.0) - x * y + jnp.log1p(jnp.exp(-jnp.abs(x)))


_loss_call = pl.pallas_call(
    _loss_body,
    grid=(RA // BA,),
    in_specs=[
        pl.BlockSpec((BA, CA), lambda i: (i, 0)),
        pl.BlockSpec((BA, CA), lambda i: (i, 0)),
    ],
    out_specs=pl.BlockSpec((BA, CA), lambda i: (i, 0)),
    out_shape=jax.ShapeDtypeStruct((RA, CA), jnp.float32),
)


STR = HBINS // NS


def _zero_hist(hist_v):
    zero16 = jnp.zeros((16,), jnp.int32)

    def _z(i, _):
        hist_v[pl.ds(i * 16, 16)] = zero16
        return 0

    lax.fori_loop(0, HBINS // 16, _z, 0)


def _hist_stream(loss_ref, base, d0, d1, sem0, sem1, process):
    pltpu.async_copy(loss_ref.at[pl.ds(base, CROWS), :], d0, sem0)
    pltpu.async_copy(loss_ref.at[pl.ds(base + CROWS, CROWS), :], d1, sem1)

    def _pair(g, _):
        off = base + 2 * g * CROWS
        pltpu.make_async_copy(loss_ref.at[pl.ds(off, CROWS), :], d0, sem0).wait()
        process(d0)
        pltpu.async_copy(loss_ref.at[pl.ds(off + 2 * CROWS, CROWS), :], d0, sem0)
        pltpu.make_async_copy(
            loss_ref.at[pl.ds(off + CROWS, CROWS), :], d1, sem1
        ).wait()
        process(d1)
        pltpu.async_copy(loss_ref.at[pl.ds(off + 3 * CROWS, CROWS), :], d1, sem1)
        return 0

    lax.fori_loop(0, NCH // 2 - 1, _pair, 0)
    tail = base + (NCH - 2) * CROWS
    pltpu.make_async_copy(loss_ref.at[pl.ds(tail, CROWS), :], d0, sem0).wait()
    process(d0)
    pltpu.make_async_copy(
        loss_ref.at[pl.ds(tail + CROWS, CROWS), :], d1, sem1
    ).wait()
    process(d1)


def _hist_hi_body(loss_ref, out_ref, d0, d1, hist_v, sem0, sem1):
    c = lax.axis_index("c")
    s = lax.axis_index("s")
    wid = s * NC + c
    _zero_hist(hist_v)

    ones = jnp.ones((16,), jnp.int32)

    def _process(buf):
        @plsc.parallel_loop(0, CHUNK // 16, unroll=8)
        def _vec(j):
            r = lax.shift_right_logical(j, 6)
            cidx = lax.bitwise_and(j, 63) * 16
            v = plsc.bitcast(buf[r, pl.ds(cidx, 16)], jnp.int32)
            plsc.addupdate_scatter(
                hist_v, [lax.shift_right_logical(v, 16)], ones
            )

    _hist_stream(loss_ref, wid * TROWS, d0, d1, sem0, sem1, _process)
    pltpu.sync_copy(hist_v, out_ref.at[wid])


_hist_hi = pl.kernel(
    _hist_hi_body,
    out_type=jax.ShapeDtypeStruct((NW, HBINS), jnp.int32),
    mesh=_mesh(),
    compiler_params=_SC_PARAMS,
    scratch_types=[
        pltpu.VMEM((CROWS, CA), jnp.float32),
        pltpu.VMEM((CROWS, CA), jnp.float32),
        pltpu.VMEM((HBINS,), jnp.int32),
        pltpu.SemaphoreType.DMA,
        pltpu.SemaphoreType.DMA,
    ],
)


def _hist_lo_body(loss_ref, outc_ref, out_ref, d0, d1, hist_v, meta_v, sem0, sem1):
    c = lax.axis_index("c")
    s = lax.axis_index("s")
    wid = s * NC + c
    _zero_hist(hist_v)
    pltpu.sync_copy(outc_ref.at[pl.ds(0, 16)], meta_v)
    b_vec = meta_v[pl.ds(0, 16)]

    ones = jnp.ones((16,), jnp.int32)

    def _process(buf):
        @plsc.parallel_loop(0, CHUNK // 16, unroll=8)
        def _vec(j):
            r = lax.shift_right_logical(j, 6)
            cidx = lax.bitwise_and(j, 63) * 16
            v = plsc.bitcast(buf[r, pl.ds(cidx, 16)], jnp.int32)
            hi = lax.shift_right_logical(v, 16)
            lo = lax.bitwise_and(v, 0xFFFF)
            plsc.addupdate_scatter(hist_v, [lo], ones, mask=hi == b_vec)

    _hist_stream(loss_ref, wid * TROWS, d0, d1, sem0, sem1, _process)
    pltpu.sync_copy(hist_v, out_ref.at[wid])


_hist_lo = pl.kernel(
    _hist_lo_body,
    out_type=jax.ShapeDtypeStruct((NW, HBINS), jnp.int32),
    mesh=_mesh(),
    compiler_params=_SC_PARAMS,
    scratch_types=[
        pltpu.VMEM((CROWS, CA), jnp.float32),
        pltpu.VMEM((CROWS, CA), jnp.float32),
        pltpu.VMEM((HBINS,), jnp.int32),
        pltpu.VMEM((16,), jnp.int32),
        pltpu.SemaphoreType.DMA,
        pltpu.SemaphoreType.DMA,
    ],
)


def _reduce_stripes(hists_ref, s, acc_v, tmp_v, shared):
    sb = s * STR
    pltpu.sync_copy(hists_ref.at[0, pl.ds(sb, STR)], acc_v)

    def _slot(k, _):
        pltpu.sync_copy(hists_ref.at[k, pl.ds(sb, STR)], tmp_v)

        def _add(j, _):
            acc_v[pl.ds(j * 16, 16)] += tmp_v[pl.ds(j * 16, 16)]
            return 0

        return lax.fori_loop(0, STR // 16, _add, 0)

    lax.fori_loop(1, NW, _slot, 0)
    pltpu.sync_copy(acc_v, shared.at[pl.ds(sb, STR)])


def _scan_combined(t_vec, tmp_v, shared):
    fifteen = jnp.full((16,), 15, jnp.int32)

    def _chunk(k, carry):
        pltpu.sync_copy(shared.at[pl.ds(k * STR, STR)], tmp_v)

        def _vec(j, carry2):
            total, found, b, pb, hb = carry2
            v = tmp_v[pl.ds(j * 16, 16)]
            sc = plsc.cumsum(v)
            cc = sc + total
            ge = cc >= t_vec
            npc = plsc.all_reduce_population_count(ge)
            anyv = npc > 0
            ffs = plsc.all_reduce_ffs(ge)
            idx_safe = jnp.minimum(ffs, fifteen)
            event = jnp.logical_and(anyv, jnp.logical_not(found))
            binbase = (k * (STR // 16) + j) * 16
            b = jnp.where(event, ffs + binbase, b)
            pb = jnp.where(event, _take16(cc, idx_safe), pb)
            hb = jnp.where(event, _take16(v, idx_safe), hb)
            found = jnp.logical_or(found, anyv)
            total = _take16(cc, fifteen)
            return total, found, b, pb, hb

        return lax.fori_loop(0, STR // 16, _vec, carry)

    zi = jnp.zeros((16,), jnp.int32)
    init = (zi, jnp.zeros((16,), jnp.bool_), zi, zi, zi)
    total, found, b, pb, hb = lax.fori_loop(0, NS, _chunk, init)
    return b, pb, hb


def _scan_hi_body(hists_ref, out_ref, acc_v, tmp_v, stage_v, shared):
    c = lax.axis_index("c")
    s = lax.axis_index("s")

    @pl.when(c == 0)
    def _():
        _reduce_stripes(hists_ref, s, acc_v, tmp_v, shared)

    plsc.subcore_barrier()

    @pl.when(jnp.logical_and(c == 0, s == 0))
    def _():
        t_vec = jnp.full((16,), TGT_HI, jnp.int32)
        b, pb, hb = _scan_combined(t_vec, tmp_v, shared)
        tp = hb - pb + t_vec
        stage_v[pl.ds(0, 16)] = b
        stage_v[pl.ds(16, 16)] = tp
        pltpu.sync_copy(stage_v, out_ref)


_scan_hi = pl.kernel(
    _scan_hi_body,
    out_type=jax.ShapeDtypeStruct((2 * 16,), jnp.int32),
    mesh=_mesh(),
    compiler_params=_SC_PARAMS,
    scratch_types=[
        pltpu.VMEM((STR,), jnp.int32),
        pltpu.VMEM((STR,), jnp.int32),
        pltpu.VMEM((2 * 16,), jnp.int32),
        pltpu.VMEM_SHARED((HBINS,), jnp.int32),
    ],
)


def _scan_lo_body(hists_ref, outc_ref, out_ref, acc_v, tmp_v, meta_v, stage_v, shared):
    c = lax.axis_index("c")
    s = lax.axis_index("s")

    @pl.when(c == 0)
    def _():
        _reduce_stripes(hists_ref, s, acc_v, tmp_v, shared)

    plsc.subcore_barrier()

    @pl.when(jnp.logical_and(c == 0, s == 0))
    def _():
        pltpu.sync_copy(outc_ref, meta_v)
        b_vec = meta_v[pl.ds(0, 16)]
        t_vec = meta_v[pl.ds(16, 16)]
        l, _pb, _hb = _scan_combined(t_vec, tmp_v, shared)
        bits = jnp.bitwise_or(jnp.left_shift(b_vec, 16), l)
        stage_v[pl.ds(0, 16)] = plsc.bitcast(bits, jnp.float32)
        pltpu.sync_copy(stage_v, out_ref)


_scan_lo = pl.kernel(
    _scan_lo_body,
    out_type=jax.ShapeDtypeStruct((16,), jnp.float32),
    mesh=_mesh(),
    compiler_params=_SC_PARAMS,
    scratch_types=[
        pltpu.VMEM((STR,), jnp.int32),
        pltpu.VMEM((STR,), jnp.int32),
        pltpu.VMEM((2 * 16,), jnp.int32),
        pltpu.VMEM((16,), jnp.float32),
        pltpu.VMEM_SHARED((HBINS,), jnp.int32),
    ],
)


def _reduce_body(t_ref, loss_ref, o_ref, acc_s, acc_c):
    i = pl.program_id(0)
    t = t_ref[0, 0]
    blk = loss_ref[...]
    m = blk > t
    ps = jnp.sum(jnp.where(m, blk, 0.0), axis=1)
    pc = jnp.sum(m.astype(jnp.float32), axis=1)

    @pl.when(i == 0)
    def _():
        acc_s[...] = ps
        acc_c[...] = pc

    @pl.when(i > 0)
    def _():
        acc_s[...] += ps
        acc_c[...] += pc

    @pl.when(i == pl.num_programs(0) - 1)
    def _():
        ts = jnp.sum(acc_s[...], axis=1, keepdims=True)
        tc = jnp.sum(acc_c[...], axis=1, keepdims=True)
        o_ref[...] = jnp.broadcast_to(ts / tc, (4, HC))


_reduce_call = pl.pallas_call(
    _reduce_body,
    grid=(FR // FB,),
    in_specs=[
        pl.BlockSpec(memory_space=pltpu.SMEM),
        pl.BlockSpec((4, FB, FC), lambda i: (0, i, 0)),
    ],
    out_specs=pl.BlockSpec((4, HC), lambda i: (0, 0)),
    out_shape=jax.ShapeDtypeStruct((4, HC), jnp.float32),
    scratch_shapes=[
        pltpu.VMEM((4, HC), jnp.float32),
        pltpu.VMEM((4, HC), jnp.float32),
    ],
)


def kernel(logits, target):
    x = logits.reshape(RA, CA)
    y = target.reshape(RA, CA)
    loss = _loss_call(x, y)
    hh = _hist_hi(loss)
    outc = _scan_hi(hh)
    hl = _hist_lo(loss, outc)
    oute = _scan_lo(hl, outc)
    t_s = oute[0].reshape(1, 1)
    om = _reduce_call(t_s, loss.reshape(4, FR, FC))
    return om[:, 0]


# speedup vs baseline: 59.3288x; 1.0145x over previous
import functools

import jax
import jax.numpy as jnp
from jax import lax
from jax.experimental import pallas as pl
from jax.experimental.pallas import tpu as pltpu
from jax.experimental.pallas import tpu_sc as plsc

N = 4 * 19 * 512 * 512
RIDX = min(int(0.001 * N), N - 1)
TGT_HI = N - RIDX

RA, CA = 19456, 1024
BA = 1024

NC, NS = 2, 16
NW = NC * NS
PER_TILE = N // NW
TROWS = RA // NW
CROWS = 8
CHUNK = CROWS * CA
NCH = TROWS // CROWS
HR, HC = 512, 128
HBINS = HR * HC

RB = 608
NB = RA // 4 // RB


def _take16(v, idx):
    return lax.gather(
        v,
        idx[:, None],
        lax.GatherDimensionNumbers(
            offset_dims=(), collapsed_slice_dims=(0,), start_index_map=(0,)
        ),
        slice_sizes=(1,),
        mode=lax.GatherScatterMode.PROMISE_IN_BOUNDS,
    )


_SC_PARAMS = pltpu.CompilerParams(
    use_tc_tiling_on_sc=False, needs_layout_passes=False
)


def _mesh():
    return plsc.VectorSubcoreMesh(
        core_axis_name="c", subcore_axis_name="s", num_cores=NC, num_subcores=NS
    )


def _loss_body(x_ref, y_ref, o_ref):
    x = x_ref[...]
    y = y_ref[...]
    o_ref[...] = jnp.maximum(x, 0.0) - x * y + jnp.log1p(jnp.exp(-jnp.abs(x)))


_loss_call = pl.pallas_call(
    _loss_body,
    grid=(RA // BA,),
    in_specs=[
        pl.BlockSpec((BA, CA), lambda i: (i, 0)),
        pl.BlockSpec((BA, CA), lambda i: (i, 0)),
    ],
    out_specs=pl.BlockSpec((BA, CA), lambda i: (i, 0)),
    out_shape=jax.ShapeDtypeStruct((RA, CA), jnp.float32),
)


STR = HBINS // NS


def _zero_hist(hist_v):
    zero16 = jnp.zeros((16,), jnp.int32)

    def _z(i, _):
        hist_v[pl.ds(i * 16, 16)] = zero16
        return 0

    lax.fori_loop(0, HBINS // 16, _z, 0)


def _hist_stream(loss_ref, base, d0, d1, sem0, sem1, process):
    pltpu.async_copy(loss_ref.at[pl.ds(base, CROWS), :], d0, sem0)
    pltpu.async_copy(loss_ref.at[pl.ds(base + CROWS, CROWS), :], d1, sem1)

    def _pair(g, _):
        off = base + 2 * g * CROWS
        pltpu.make_async_copy(loss_ref.at[pl.ds(off, CROWS), :], d0, sem0).wait()
        process(d0)
        pltpu.async_copy(loss_ref.at[pl.ds(off + 2 * CROWS, CROWS), :], d0, sem0)
        pltpu.make_async_copy(
            loss_ref.at[pl.ds(off + CROWS, CROWS), :], d1, sem1
        ).wait()
        process(d1)
        pltpu.async_copy(loss_ref.at[pl.ds(off + 3 * CROWS, CROWS), :], d1, sem1)
        return 0

    lax.fori_loop(0, NCH // 2 - 1, _pair, 0)
    tail = base + (NCH - 2) * CROWS
    pltpu.make_async_copy(loss_ref.at[pl.ds(tail, CROWS), :], d0, sem0).wait()
    process(d0)
    pltpu.make_async_copy(
        loss_ref.at[pl.ds(tail + CROWS, CROWS), :], d1, sem1
    ).wait()
    process(d1)


def _hist_hi_body(loss_ref, out_ref, d0, d1, hist_v, sem0, sem1):
    c = lax.axis_index("c")
    s = lax.axis_index("s")
    wid = s * NC + c
    _zero_hist(hist_v)

    ones = jnp.ones((16,), jnp.int32)

    def _process(buf):
        @plsc.parallel_loop(0, CHUNK // 16, unroll=8)
        def _vec(j):
            r = lax.shift_right_logical(j, 6)
            cidx = lax.bitwise_and(j, 63) * 16
            v = plsc.bitcast(buf[r, pl.ds(cidx, 16)], jnp.int32)
            plsc.addupdate_scatter(
                hist_v, [lax.shift_right_logical(v, 16)], ones
            )

    _hist_stream(loss_ref, wid * TROWS, d0, d1, sem0, sem1, _process)
    pltpu.sync_copy(hist_v, out_ref.at[wid])


_hist_hi = pl.kernel(
    _hist_hi_body,
    out_type=jax.ShapeDtypeStruct((NW, HBINS), jnp.int32),
    mesh=_mesh(),
    compiler_params=_SC_PARAMS,
    scratch_types=[
        pltpu.VMEM((CROWS, CA), jnp.float32),
        pltpu.VMEM((CROWS, CA), jnp.float32),
        pltpu.VMEM((HBINS,), jnp.int32),
        pltpu.SemaphoreType.DMA,
        pltpu.SemaphoreType.DMA,
    ],
)


def _hist_lo_body(loss_ref, outc_ref, out_ref, d0, d1, hist_v, meta_v, sem0, sem1):
    c = lax.axis_index("c")
    s = lax.axis_index("s")
    wid = s * NC + c
    _zero_hist(hist_v)
    pltpu.sync_copy(outc_ref.at[pl.ds(0, 16)], meta_v)
    b_vec = meta_v[pl.ds(0, 16)]

    ones = jnp.ones((16,), jnp.int32)

    def _process(buf):
        @plsc.parallel_loop(0, CHUNK // 16, unroll=8)
        def _vec(j):
            r = lax.shift_right_logical(j, 6)
            cidx = lax.bitwise_and(j, 63) * 16
            v = plsc.bitcast(buf[r, pl.ds(cidx, 16)], jnp.int32)
            hi = lax.shift_right_logical(v, 16)
            lo = lax.bitwise_and(v, 0xFFFF)
            plsc.addupdate_scatter(hist_v, [lo], ones, mask=hi == b_vec)

    _hist_stream(loss_ref, wid * TROWS, d0, d1, sem0, sem1, _process)
    pltpu.sync_copy(hist_v, out_ref.at[wid])


_hist_lo = pl.kernel(
    _hist_lo_body,
    out_type=jax.ShapeDtypeStruct((NW, HBINS), jnp.int32),
    mesh=_mesh(),
    compiler_params=_SC_PARAMS,
    scratch_types=[
        pltpu.VMEM((CROWS, CA), jnp.float32),
        pltpu.VMEM((CROWS, CA), jnp.float32),
        pltpu.VMEM((HBINS,), jnp.int32),
        pltpu.VMEM((16,), jnp.int32),
        pltpu.SemaphoreType.DMA,
        pltpu.SemaphoreType.DMA,
    ],
)


def _reduce_stripes(hists_ref, s, acc_v, tmp_v, shared):
    sb = s * STR
    pltpu.sync_copy(hists_ref.at[0, pl.ds(sb, STR)], acc_v)

    def _slot(k, _):
        pltpu.sync_copy(hists_ref.at[k, pl.ds(sb, STR)], tmp_v)

        def _add(j, _):
            acc_v[pl.ds(j * 16, 16)] += tmp_v[pl.ds(j * 16, 16)]
            return 0

        return lax.fori_loop(0, STR // 16, _add, 0)

    lax.fori_loop(1, NW, _slot, 0)
    pltpu.sync_copy(acc_v, shared.at[pl.ds(sb, STR)])


def _scan_combined(t_vec, tmp_v, shared):
    fifteen = jnp.full((16,), 15, jnp.int32)

    def _chunk(k, carry):
        pltpu.sync_copy(shared.at[pl.ds(k * STR, STR)], tmp_v)

        def _vec(j, carry2):
            total, found, b, pb, hb = carry2
            v = tmp_v[pl.ds(j * 16, 16)]
            sc = plsc.cumsum(v)
            cc = sc + total
            ge = cc >= t_vec
            npc = plsc.all_reduce_population_count(ge)
            anyv = npc > 0
            ffs = plsc.all_reduce_ffs(ge)
            idx_safe = jnp.minimum(ffs, fifteen)
            event = jnp.logical_and(anyv, jnp.logical_not(found))
            binbase = (k * (STR // 16) + j) * 16
            b = jnp.where(event, ffs + binbase, b)
            pb = jnp.where(event, _take16(cc, idx_safe), pb)
            hb = jnp.where(event, _take16(v, idx_safe), hb)
            found = jnp.logical_or(found, anyv)
            total = _take16(cc, fifteen)
            return total, found, b, pb, hb

        return lax.fori_loop(0, STR // 16, _vec, carry)

    zi = jnp.zeros((16,), jnp.int32)
    init = (zi, jnp.zeros((16,), jnp.bool_), zi, zi, zi)
    total, found, b, pb, hb = lax.fori_loop(0, NS, _chunk, init)
    return b, pb, hb


def _scan_hi_body(hists_ref, out_ref, acc_v, tmp_v, stage_v, shared):
    c = lax.axis_index("c")
    s = lax.axis_index("s")

    @pl.when(c == 0)
    def _():
        _reduce_stripes(hists_ref, s, acc_v, tmp_v, shared)

    plsc.subcore_barrier()

    @pl.when(jnp.logical_and(c == 0, s == 0))
    def _():
        t_vec = jnp.full((16,), TGT_HI, jnp.int32)
        b, pb, hb = _scan_combined(t_vec, tmp_v, shared)
        tp = hb - pb + t_vec
        stage_v[pl.ds(0, 16)] = b
        stage_v[pl.ds(16, 16)] = tp
        pltpu.sync_copy(stage_v, out_ref)


_scan_hi = pl.kernel(
    _scan_hi_body,
    out_type=jax.ShapeDtypeStruct((2 * 16,), jnp.int32),
    mesh=_mesh(),
    compiler_params=_SC_PARAMS,
    scratch_types=[
        pltpu.VMEM((STR,), jnp.int32),
        pltpu.VMEM((STR,), jnp.int32),
        pltpu.VMEM((2 * 16,), jnp.int32),
        pltpu.VMEM_SHARED((HBINS,), jnp.int32),
    ],
)


def _scan_lo_body(hists_ref, outc_ref, out_ref, acc_v, tmp_v, meta_v, stage_v, shared):
    c = lax.axis_index("c")
    s = lax.axis_index("s")

    @pl.when(c == 0)
    def _():
        _reduce_stripes(hists_ref, s, acc_v, tmp_v, shared)

    plsc.subcore_barrier()

    @pl.when(jnp.logical_and(c == 0, s == 0))
    def _():
        pltpu.sync_copy(outc_ref, meta_v)
        b_vec = meta_v[pl.ds(0, 16)]
        t_vec = meta_v[pl.ds(16, 16)]
        l, _pb, _hb = _scan_combined(t_vec, tmp_v, shared)
        bits = jnp.bitwise_or(jnp.left_shift(b_vec, 16), l)
        stage_v[pl.ds(0, 16)] = plsc.bitcast(bits, jnp.float32)
        pltpu.sync_copy(stage_v, out_ref)


_scan_lo = pl.kernel(
    _scan_lo_body,
    out_type=jax.ShapeDtypeStruct((16,), jnp.float32),
    mesh=_mesh(),
    compiler_params=_SC_PARAMS,
    scratch_types=[
        pltpu.VMEM((STR,), jnp.int32),
        pltpu.VMEM((STR,), jnp.int32),
        pltpu.VMEM((2 * 16,), jnp.int32),
        pltpu.VMEM((16,), jnp.float32),
        pltpu.VMEM_SHARED((HBINS,), jnp.int32),
    ],
)


def _reduce_body(t_ref, loss_ref, o_ref, acc_s, acc_c):
    j = pl.program_id(1)
    t = t_ref[0, 0]
    blk = loss_ref[...]
    m = blk > t
    ps = jnp.sum(jnp.where(m, blk, 0.0), axis=0, keepdims=True)
    pc = jnp.sum(m.astype(jnp.float32), axis=0, keepdims=True)

    @pl.when(j == 0)
    def _():
        acc_s[...] = ps
        acc_c[...] = pc

    @pl.when(j > 0)
    def _():
        acc_s[...] += ps
        acc_c[...] += pc

    @pl.when(j == pl.num_programs(1) - 1)
    def _():
        ts = jnp.sum(acc_s[...])
        tc = jnp.sum(acc_c[...])
        b = pl.program_id(0)
        o_ref[pl.ds(b, 1), :] = jnp.full((1, HC), ts / tc, jnp.float32)


_reduce_call = pl.pallas_call(
    _reduce_body,
    grid=(4, NB),
    in_specs=[
        pl.BlockSpec(memory_space=pltpu.SMEM),
        pl.BlockSpec((RB, CA), lambda b, j: (b * NB + j, 0)),
    ],
    out_specs=pl.BlockSpec((4, HC), lambda b, j: (0, 0)),
    out_shape=jax.ShapeDtypeStruct((4, HC), jnp.float32),
    scratch_shapes=[
        pltpu.VMEM((1, CA), jnp.float32),
        pltpu.VMEM((1, CA), jnp.float32),
    ],
)


def kernel(logits, target):
    x = logits.reshape(RA, CA)
    y = target.reshape(RA, CA)
    loss = _loss_call(x, y)
    hh = _hist_hi(loss)
    outc = _scan_hi(hh)
    hl = _hist_lo(loss, outc)
    oute = _scan_lo(hl, outc)
    t_s = oute[0].reshape(1, 1)
    om = _reduce_call(t_s, loss)
    return om[:, 0]


# speedup vs baseline: 77.9840x; 1.3144x over previous
import functools

import jax
import jax.numpy as jnp
from jax import lax
from jax.experimental import pallas as pl
from jax.experimental.pallas import tpu as pltpu
from jax.experimental.pallas import tpu_sc as plsc

N = 4 * 19 * 512 * 512
RIDX = min(int(0.001 * N), N - 1)
TGT_HI = N - RIDX

RA, CA = 38912, 512
BA = 1024

NC, NS = 2, 16
NW = NC * NS
PER_TILE = N // NW
TROWS = RA // NW
CROWS = 16
CHUNK = CROWS * CA
NCH = TROWS // CROWS
HR, HC = 512, 128
HBINS = HR * HC

RB = 1216
NB = RA // 4 // RB


def _take16(v, idx):
    return lax.gather(
        v,
        idx[:, None],
        lax.GatherDimensionNumbers(
            offset_dims=(), collapsed_slice_dims=(0,), start_index_map=(0,)
        ),
        slice_sizes=(1,),
        mode=lax.GatherScatterMode.PROMISE_IN_BOUNDS,
    )


_SC_PARAMS = pltpu.CompilerParams(
    use_tc_tiling_on_sc=False, needs_layout_passes=False
)


def _mesh():
    return plsc.VectorSubcoreMesh(
        core_axis_name="c", subcore_axis_name="s", num_cores=NC, num_subcores=NS
    )


def _loss_body(x_ref, y_ref, o_ref):
    x = x_ref[...]
    y = y_ref[...]
    o_ref[...] = jnp.maximum(x, 0.0) - x * y + jnp.log1p(jnp.exp(-jnp.abs(x)))


_loss_call = pl.pallas_call(
    _loss_body,
    grid=(RA // BA,),
    in_specs=[
        pl.BlockSpec((BA, CA), lambda i: (i, 0)),
        pl.BlockSpec((BA, CA), lambda i: (i, 0)),
    ],
    out_specs=pl.BlockSpec((BA, CA), lambda i: (i, 0)),
    out_shape=jax.ShapeDtypeStruct((RA, CA), jnp.float32),
)


STR = HBINS // NS


def _zero_hist(hist_v):
    zero16 = jnp.zeros((16,), jnp.int32)

    def _z(i, _):
        hist_v[pl.ds(i * 16, 16)] = zero16
        return 0

    lax.fori_loop(0, HBINS // 16, _z, 0)


def _hist_stream(loss_ref, base, d0, d1, sem0, sem1, process):
    pltpu.async_copy(loss_ref.at[pl.ds(base, CROWS), :], d0, sem0)
    pltpu.async_copy(loss_ref.at[pl.ds(base + CROWS, CROWS), :], d1, sem1)

    def _pair(g, _):
        off = base + 2 * g * CROWS
        pltpu.make_async_copy(loss_ref.at[pl.ds(off, CROWS), :], d0, sem0).wait()
        process(d0)
        pltpu.async_copy(loss_ref.at[pl.ds(off + 2 * CROWS, CROWS), :], d0, sem0)
        pltpu.make_async_copy(
            loss_ref.at[pl.ds(off + CROWS, CROWS), :], d1, sem1
        ).wait()
        process(d1)
        pltpu.async_copy(loss_ref.at[pl.ds(off + 3 * CROWS, CROWS), :], d1, sem1)
        return 0

    lax.fori_loop(0, NCH // 2 - 1, _pair, 0)
    tail = base + (NCH - 2) * CROWS
    pltpu.make_async_copy(loss_ref.at[pl.ds(tail, CROWS), :], d0, sem0).wait()
    process(d0)
    pltpu.make_async_copy(
        loss_ref.at[pl.ds(tail + CROWS, CROWS), :], d1, sem1
    ).wait()
    process(d1)


def _hist_hi_body(loss_ref, out_ref, d0, d1, hist_v, sem0, sem1):
    c = lax.axis_index("c")
    s = lax.axis_index("s")
    wid = s * NC + c
    _zero_hist(hist_v)

    ones = jnp.ones((16,), jnp.int32)

    def _process(buf):
        @plsc.parallel_loop(0, CHUNK // 16, unroll=8)
        def _vec(j):
            r = lax.shift_right_logical(j, 5)
            cidx = lax.bitwise_and(j, 31) * 16
            v = plsc.bitcast(buf[r, pl.ds(cidx, 16)], jnp.int32)
            plsc.addupdate_scatter(
                hist_v, [lax.shift_right_logical(v, 16)], ones
            )

    _hist_stream(loss_ref, wid * TROWS, d0, d1, sem0, sem1, _process)
    pltpu.sync_copy(hist_v, out_ref.at[wid])


_hist_hi = pl.kernel(
    _hist_hi_body,
    out_type=jax.ShapeDtypeStruct((NW, HBINS), jnp.int32),
    mesh=_mesh(),
    compiler_params=_SC_PARAMS,
    scratch_types=[
        pltpu.VMEM((CROWS, CA), jnp.float32),
        pltpu.VMEM((CROWS, CA), jnp.float32),
        pltpu.VMEM((HBINS,), jnp.int32),
        pltpu.SemaphoreType.DMA,
        pltpu.SemaphoreType.DMA,
    ],
)


def _hist_lo_body(loss_ref, outc_ref, out_ref, d0, d1, hist_v, meta_v, sem0, sem1):
    c = lax.axis_index("c")
    s = lax.axis_index("s")
    wid = s * NC + c
    _zero_hist(hist_v)
    pltpu.sync_copy(outc_ref.at[pl.ds(0, 16)], meta_v)
    b_vec = meta_v[pl.ds(0, 16)]

    ones = jnp.ones((16,), jnp.int32)

    def _process(buf):
        @plsc.parallel_loop(0, CHUNK // 16, unroll=8)
        def _vec(j):
            r = lax.shift_right_logical(j, 5)
            cidx = lax.bitwise_and(j, 31) * 16
            v = plsc.bitcast(buf[r, pl.ds(cidx, 16)], jnp.int32)
            hi = lax.shift_right_logical(v, 16)
            lo = lax.bitwise_and(v, 0xFFFF)
            plsc.addupdate_scatter(hist_v, [lo], ones, mask=hi == b_vec)

    _hist_stream(loss_ref, wid * TROWS, d0, d1, sem0, sem1, _process)
    pltpu.sync_copy(hist_v, out_ref.at[wid])


_hist_lo = pl.kernel(
    _hist_lo_body,
    out_type=jax.ShapeDtypeStruct((NW, HBINS), jnp.int32),
    mesh=_mesh(),
    compiler_params=_SC_PARAMS,
    scratch_types=[
        pltpu.VMEM((CROWS, CA), jnp.float32),
        pltpu.VMEM((CROWS, CA), jnp.float32),
        pltpu.VMEM((HBINS,), jnp.int32),
        pltpu.VMEM((16,), jnp.int32),
        pltpu.SemaphoreType.DMA,
        pltpu.SemaphoreType.DMA,
    ],
)


def _reduce_stripes(hists_ref, s, acc_v, tmp_v, shared):
    sb = s * STR
    pltpu.sync_copy(hists_ref.at[0, pl.ds(sb, STR)], acc_v)

    def _slot(k, _):
        pltpu.sync_copy(hists_ref.at[k, pl.ds(sb, STR)], tmp_v)

        def _add(j, _):
            acc_v[pl.ds(j * 16, 16)] += tmp_v[pl.ds(j * 16, 16)]
            return 0

        return lax.fori_loop(0, STR // 16, _add, 0)

    lax.fori_loop(1, NW, _slot, 0)
    pltpu.sync_copy(acc_v, shared.at[pl.ds(sb, STR)])


def _scan_combined(t_vec, tmp_v, shared):
    fifteen = jnp.full((16,), 15, jnp.int32)

    def _chunk(k, carry):
        pltpu.sync_copy(shared.at[pl.ds(k * STR, STR)], tmp_v)

        def _vec(j, carry2):
            total, found, b, pb, hb = carry2
            v = tmp_v[pl.ds(j * 16, 16)]
            sc = plsc.cumsum(v)
            cc = sc + total
            ge = cc >= t_vec
            npc = plsc.all_reduce_population_count(ge)
            anyv = npc > 0
            ffs = plsc.all_reduce_ffs(ge)
            idx_safe = jnp.minimum(ffs, fifteen)
            event = jnp.logical_and(anyv, jnp.logical_not(found))
            binbase = (k * (STR // 16) + j) * 16
            b = jnp.where(event, ffs + binbase, b)
            pb = jnp.where(event, _take16(cc, idx_safe), pb)
            hb = jnp.where(event, _take16(v, idx_safe), hb)
            found = jnp.logical_or(found, anyv)
            total = _take16(cc, fifteen)
            return total, found, b, pb, hb

        return lax.fori_loop(0, STR // 16, _vec, carry)

    zi = jnp.zeros((16,), jnp.int32)
    init = (zi, jnp.zeros((16,), jnp.bool_), zi, zi, zi)
    total, found, b, pb, hb = lax.fori_loop(0, NS, _chunk, init)
    return b, pb, hb


def _scan_hi_body(hists_ref, out_ref, acc_v, tmp_v, stage_v, shared):
    c = lax.axis_index("c")
    s = lax.axis_index("s")

    @pl.when(c == 0)
    def _():
        _reduce_stripes(hists_ref, s, acc_v, tmp_v, shared)

    plsc.subcore_barrier()

    @pl.when(jnp.logical_and(c == 0, s == 0))
    def _():
        t_vec = jnp.full((16,), TGT_HI, jnp.int32)
        b, pb, hb = _scan_combined(t_vec, tmp_v, shared)
        tp = hb - pb + t_vec
        stage_v[pl.ds(0, 16)] = b
        stage_v[pl.ds(16, 16)] = tp
        pltpu.sync_copy(stage_v, out_ref)


_scan_hi = pl.kernel(
    _scan_hi_body,
    out_type=jax.ShapeDtypeStruct((2 * 16,), jnp.int32),
    mesh=_mesh(),
    compiler_params=_SC_PARAMS,
    scratch_types=[
        pltpu.VMEM((STR,), jnp.int32),
        pltpu.VMEM((STR,), jnp.int32),
        pltpu.VMEM((2 * 16,), jnp.int32),
        pltpu.VMEM_SHARED((HBINS,), jnp.int32),
    ],
)


def _scan_lo_body(hists_ref, outc_ref, out_ref, acc_v, tmp_v, meta_v, stage_v, shared):
    c = lax.axis_index("c")
    s = lax.axis_index("s")

    @pl.when(c == 0)
    def _():
        _reduce_stripes(hists_ref, s, acc_v, tmp_v, shared)

    plsc.subcore_barrier()

    @pl.when(jnp.logical_and(c == 0, s == 0))
    def _():
        pltpu.sync_copy(outc_ref, meta_v)
        b_vec = meta_v[pl.ds(0, 16)]
        t_vec = meta_v[pl.ds(16, 16)]
        l, _pb, _hb = _scan_combined(t_vec, tmp_v, shared)
        bits = jnp.bitwise_or(jnp.left_shift(b_vec, 16), l)
        stage_v[pl.ds(0, 16)] = plsc.bitcast(bits, jnp.float32)
        pltpu.sync_copy(stage_v, out_ref)


_scan_lo = pl.kernel(
    _scan_lo_body,
    out_type=jax.ShapeDtypeStruct((16,), jnp.float32),
    mesh=_mesh(),
    compiler_params=_SC_PARAMS,
    scratch_types=[
        pltpu.VMEM((STR,), jnp.int32),
        pltpu.VMEM((STR,), jnp.int32),
        pltpu.VMEM((2 * 16,), jnp.int32),
        pltpu.VMEM((16,), jnp.float32),
        pltpu.VMEM_SHARED((HBINS,), jnp.int32),
    ],
)


def _reduce_body(t_ref, loss_ref, o_ref, acc_s, acc_c):
    j = pl.program_id(1)
    t = t_ref[0, 0]
    blk = loss_ref[...]
    m = blk > t
    ps = jnp.sum(jnp.where(m, blk, 0.0), axis=0, keepdims=True)
    pc = jnp.sum(m.astype(jnp.float32), axis=0, keepdims=True)

    @pl.when(j == 0)
    def _():
        acc_s[...] = ps
        acc_c[...] = pc

    @pl.when(j > 0)
    def _():
        acc_s[...] += ps
        acc_c[...] += pc

    @pl.when(j == pl.num_programs(1) - 1)
    def _():
        ts = jnp.sum(acc_s[...])
        tc = jnp.sum(acc_c[...])
        b = pl.program_id(0)
        o_ref[pl.ds(b, 1), :] = jnp.full((1, HC), ts / tc, jnp.float32)


_reduce_call = pl.pallas_call(
    _reduce_body,
    grid=(4, NB),
    in_specs=[
        pl.BlockSpec(memory_space=pltpu.SMEM),
        pl.BlockSpec((RB, CA), lambda b, j: (b * NB + j, 0)),
    ],
    out_specs=pl.BlockSpec((4, HC), lambda b, j: (0, 0)),
    out_shape=jax.ShapeDtypeStruct((4, HC), jnp.float32),
    scratch_shapes=[
        pltpu.VMEM((1, CA), jnp.float32),
        pltpu.VMEM((1, CA), jnp.float32),
    ],
)


def kernel(logits, target):
    x = logits.reshape(RA, CA)
    y = target.reshape(RA, CA)
    loss = _loss_call(x, y)
    hh = _hist_hi(loss)
    outc = _scan_hi(hh)
    hl = _hist_lo(loss, outc)
    oute = _scan_lo(hl, outc)
    t_s = oute[0].reshape(1, 1)
    om = _reduce_call(t_s, loss)
    return om[:, 0]


# speedup vs baseline: 86.2581x; 1.1061x over previous
import functools

import jax
import jax.numpy as jnp
from jax import lax
from jax.experimental import pallas as pl
from jax.experimental.pallas import tpu as pltpu
from jax.experimental.pallas import tpu_sc as plsc

N = 4 * 19 * 512 * 512
RIDX = min(int(0.001 * N), N - 1)
TGT_HI = N - RIDX

RA, CA = 38912, 512
BA = 1024

NC, NS = 2, 16
NW = NC * NS
PER_TILE = N // NW
TROWS = RA // NW
CROWS = 16
CHUNK = CROWS * CA
NCH = TROWS // CROWS
HR, HC = 512, 128
HBINS = HR * HC

RB = 1216
NB = RA // 4 // RB


def _take16(v, idx):
    return lax.gather(
        v,
        idx[:, None],
        lax.GatherDimensionNumbers(
            offset_dims=(), collapsed_slice_dims=(0,), start_index_map=(0,)
        ),
        slice_sizes=(1,),
        mode=lax.GatherScatterMode.PROMISE_IN_BOUNDS,
    )


_SC_PARAMS = pltpu.CompilerParams(
    use_tc_tiling_on_sc=False, needs_layout_passes=False
)


def _mesh():
    return plsc.VectorSubcoreMesh(
        core_axis_name="c", subcore_axis_name="s", num_cores=NC, num_subcores=NS
    )


def _loss_body(x_ref, y_ref, o_ref):
    x = x_ref[...]
    y = y_ref[...]
    r = jnp.maximum(x, 0.0) - x * y + jnp.log1p(jnp.exp(-jnp.abs(x)))
    o_ref[...] = r.reshape(BA * CA)


_loss_call = pl.pallas_call(
    _loss_body,
    grid=(RA // BA,),
    in_specs=[
        pl.BlockSpec((BA, CA), lambda i: (i, 0)),
        pl.BlockSpec((BA, CA), lambda i: (i, 0)),
    ],
    out_specs=pl.BlockSpec((BA * CA,), lambda i: (i,)),
    out_shape=jax.ShapeDtypeStruct((N,), jnp.float32),
)


STR = HBINS // NS


def _zero_hist(hist_v):
    zero16 = jnp.zeros((16,), jnp.int32)

    def _z(i, _):
        hist_v[pl.ds(i * 16, 16)] = zero16
        return 0

    lax.fori_loop(0, HBINS // 16, _z, 0)


def _hist_stream(loss_ref, base, d0, d1, sem0, sem1, process):
    pltpu.async_copy(loss_ref.at[pl.ds(base, CHUNK)], d0, sem0)
    pltpu.async_copy(loss_ref.at[pl.ds(base + CHUNK, CHUNK)], d1, sem1)

    def _pair(g, _):
        off = base + 2 * g * CHUNK
        pltpu.make_async_copy(loss_ref.at[pl.ds(off, CHUNK)], d0, sem0).wait()
        process(d0)
        pltpu.async_copy(loss_ref.at[pl.ds(off + 2 * CHUNK, CHUNK)], d0, sem0)
        pltpu.make_async_copy(
            loss_ref.at[pl.ds(off + CHUNK, CHUNK)], d1, sem1
        ).wait()
        process(d1)
        pltpu.async_copy(loss_ref.at[pl.ds(off + 3 * CHUNK, CHUNK)], d1, sem1)
        return 0

    lax.fori_loop(0, NCH // 2 - 1, _pair, 0)
    tail = base + (NCH - 2) * CHUNK
    pltpu.make_async_copy(loss_ref.at[pl.ds(tail, CHUNK)], d0, sem0).wait()
    process(d0)
    pltpu.make_async_copy(loss_ref.at[pl.ds(tail + CHUNK, CHUNK)], d1, sem1).wait()
    process(d1)


def _hist_hi_body(loss_ref, out_ref, d0, d1, hist_v, sem0, sem1):
    c = lax.axis_index("c")
    s = lax.axis_index("s")
    wid = s * NC + c
    _zero_hist(hist_v)

    ones = jnp.ones((16,), jnp.int32)

    def _process(buf):
        @plsc.parallel_loop(0, CHUNK // 16, unroll=8)
        def _vec(j):
            v = plsc.bitcast(buf[pl.ds(j * 16, 16)], jnp.int32)
            plsc.addupdate_scatter(
                hist_v, [lax.shift_right_logical(v, 16)], ones
            )

    _hist_stream(loss_ref, wid * PER_TILE, d0, d1, sem0, sem1, _process)
    pltpu.sync_copy(hist_v, out_ref.at[wid])


_hist_hi = pl.kernel(
    _hist_hi_body,
    out_type=jax.ShapeDtypeStruct((NW, HBINS), jnp.int32),
    mesh=_mesh(),
    compiler_params=_SC_PARAMS,
    scratch_types=[
        pltpu.VMEM((CHUNK,), jnp.float32),
        pltpu.VMEM((CHUNK,), jnp.float32),
        pltpu.VMEM((HBINS,), jnp.int32),
        pltpu.SemaphoreType.DMA,
        pltpu.SemaphoreType.DMA,
    ],
)


def _hist_lo_body(loss_ref, outc_ref, out_ref, d0, d1, hist_v, meta_v, sem0, sem1):
    c = lax.axis_index("c")
    s = lax.axis_index("s")
    wid = s * NC + c
    _zero_hist(hist_v)
    pltpu.sync_copy(outc_ref.at[pl.ds(0, 16)], meta_v)
    b_vec = meta_v[pl.ds(0, 16)]

    ones = jnp.ones((16,), jnp.int32)

    def _process(buf):
        @plsc.parallel_loop(0, CHUNK // 16, unroll=8)
        def _vec(j):
            v = plsc.bitcast(buf[pl.ds(j * 16, 16)], jnp.int32)
            hi = lax.shift_right_logical(v, 16)
            lo = lax.bitwise_and(v, 0xFFFF)
            plsc.addupdate_scatter(hist_v, [lo], ones, mask=hi == b_vec)

    _hist_stream(loss_ref, wid * PER_TILE, d0, d1, sem0, sem1, _process)
    pltpu.sync_copy(hist_v, out_ref.at[wid])


_hist_lo = pl.kernel(
    _hist_lo_body,
    out_type=jax.ShapeDtypeStruct((NW, HBINS), jnp.int32),
    mesh=_mesh(),
    compiler_params=_SC_PARAMS,
    scratch_types=[
        pltpu.VMEM((CHUNK,), jnp.float32),
        pltpu.VMEM((CHUNK,), jnp.float32),
        pltpu.VMEM((HBINS,), jnp.int32),
        pltpu.VMEM((16,), jnp.int32),
        pltpu.SemaphoreType.DMA,
        pltpu.SemaphoreType.DMA,
    ],
)


def _reduce_stripes(hists_ref, s, acc_v, tmp_v, shared):
    sb = s * STR
    pltpu.sync_copy(hists_ref.at[0, pl.ds(sb, STR)], acc_v)

    def _slot(k, _):
        pltpu.sync_copy(hists_ref.at[k, pl.ds(sb, STR)], tmp_v)

        def _add(j, _):
            acc_v[pl.ds(j * 16, 16)] += tmp_v[pl.ds(j * 16, 16)]
            return 0

        return lax.fori_loop(0, STR // 16, _add, 0)

    lax.fori_loop(1, NW, _slot, 0)
    pltpu.sync_copy(acc_v, shared.at[pl.ds(sb, STR)])


def _scan_combined(t_vec, tmp_v, shared):
    fifteen = jnp.full((16,), 15, jnp.int32)

    def _chunk(k, carry):
        pltpu.sync_copy(shared.at[pl.ds(k * STR, STR)], tmp_v)

        def _vec(j, carry2):
            total, found, b, pb, hb = carry2
            v = tmp_v[pl.ds(j * 16, 16)]
            sc = plsc.cumsum(v)
            cc = sc + total
            ge = cc >= t_vec
            npc = plsc.all_reduce_population_count(ge)
            anyv = npc > 0
            ffs = plsc.all_reduce_ffs(ge)
            idx_safe = jnp.minimum(ffs, fifteen)
            event = jnp.logical_and(anyv, jnp.logical_not(found))
            binbase = (k * (STR // 16) + j) * 16
            b = jnp.where(event, ffs + binbase, b)
            pb = jnp.where(event, _take16(cc, idx_safe), pb)
            hb = jnp.where(event, _take16(v, idx_safe), hb)
            found = jnp.logical_or(found, anyv)
            total = _take16(cc, fifteen)
            return total, found, b, pb, hb

        return lax.fori_loop(0, STR // 16, _vec, carry)

    zi = jnp.zeros((16,), jnp.int32)
    init = (zi, jnp.zeros((16,), jnp.bool_), zi, zi, zi)
    total, found, b, pb, hb = lax.fori_loop(0, NS, _chunk, init)
    return b, pb, hb


def _scan_hi_body(hists_ref, out_ref, acc_v, tmp_v, stage_v, shared):
    c = lax.axis_index("c")
    s = lax.axis_index("s")

    @pl.when(c == 0)
    def _():
        _reduce_stripes(hists_ref, s, acc_v, tmp_v, shared)

    plsc.subcore_barrier()

    @pl.when(jnp.logical_and(c == 0, s == 0))
    def _():
        t_vec = jnp.full((16,), TGT_HI, jnp.int32)
        b, pb, hb = _scan_combined(t_vec, tmp_v, shared)
        tp = hb - pb + t_vec
        stage_v[pl.ds(0, 16)] = b
        stage_v[pl.ds(16, 16)] = tp
        pltpu.sync_copy(stage_v, out_ref)


_scan_hi = pl.kernel(
    _scan_hi_body,
    out_type=jax.ShapeDtypeStruct((2 * 16,), jnp.int32),
    mesh=_mesh(),
    compiler_params=_SC_PARAMS,
    scratch_types=[
        pltpu.VMEM((STR,), jnp.int32),
        pltpu.VMEM((STR,), jnp.int32),
        pltpu.VMEM((2 * 16,), jnp.int32),
        pltpu.VMEM_SHARED((HBINS,), jnp.int32),
    ],
)


def _scan_lo_body(hists_ref, outc_ref, out_ref, acc_v, tmp_v, meta_v, stage_v, shared):
    c = lax.axis_index("c")
    s = lax.axis_index("s")

    @pl.when(c == 0)
    def _():
        _reduce_stripes(hists_ref, s, acc_v, tmp_v, shared)

    plsc.subcore_barrier()

    @pl.when(jnp.logical_and(c == 0, s == 0))
    def _():
        pltpu.sync_copy(outc_ref, meta_v)
        b_vec = meta_v[pl.ds(0, 16)]
        t_vec = meta_v[pl.ds(16, 16)]
        l, _pb, _hb = _scan_combined(t_vec, tmp_v, shared)
        bits = jnp.bitwise_or(jnp.left_shift(b_vec, 16), l)
        stage_v[pl.ds(0, 16)] = plsc.bitcast(bits, jnp.float32)
        pltpu.sync_copy(stage_v, out_ref)


_scan_lo = pl.kernel(
    _scan_lo_body,
    out_type=jax.ShapeDtypeStruct((16,), jnp.float32),
    mesh=_mesh(),
    compiler_params=_SC_PARAMS,
    scratch_types=[
        pltpu.VMEM((STR,), jnp.int32),
        pltpu.VMEM((STR,), jnp.int32),
        pltpu.VMEM((2 * 16,), jnp.int32),
        pltpu.VMEM((16,), jnp.float32),
        pltpu.VMEM_SHARED((HBINS,), jnp.int32),
    ],
)


def _reduce_body(t_ref, loss_ref, o_ref, acc_s, acc_c):
    j = pl.program_id(1)
    t = t_ref[0, 0]
    blk = loss_ref[...].reshape(RB * CA // HC, HC)
    m = blk > t
    ps = jnp.sum(jnp.where(m, blk, 0.0), axis=0, keepdims=True)
    pc = jnp.sum(m.astype(jnp.float32), axis=0, keepdims=True)

    @pl.when(j == 0)
    def _():
        acc_s[...] = ps
        acc_c[...] = pc

    @pl.when(j > 0)
    def _():
        acc_s[...] += ps
        acc_c[...] += pc

    @pl.when(j == pl.num_programs(1) - 1)
    def _():
        ts = jnp.sum(acc_s[...])
        tc = jnp.sum(acc_c[...])
        b = pl.program_id(0)
        o_ref[pl.ds(b, 1), :] = jnp.full((1, HC), ts / tc, jnp.float32)


_reduce_call = pl.pallas_call(
    _reduce_body,
    grid=(4, NB),
    in_specs=[
        pl.BlockSpec(memory_space=pltpu.SMEM),
        pl.BlockSpec((RB * CA,), lambda b, j: (b * NB + j,)),
    ],
    out_specs=pl.BlockSpec((4, HC), lambda b, j: (0, 0)),
    out_shape=jax.ShapeDtypeStruct((4, HC), jnp.float32),
    scratch_shapes=[
        pltpu.VMEM((1, HC), jnp.float32),
        pltpu.VMEM((1, HC), jnp.float32),
    ],
)


def kernel(logits, target):
    x = logits.reshape(RA, CA)
    y = target.reshape(RA, CA)
    loss = _loss_call(x, y)
    hh = _hist_hi(loss)
    outc = _scan_hi(hh)
    hl = _hist_lo(loss, outc)
    oute = _scan_lo(hl, outc)
    t_s = oute[0].reshape(1, 1)
    om = _reduce_call(t_s, loss)
    return om[:, 0]
